# Initial kernel scaffold; baseline (speedup 1.0000x reference)
import jax
import jax.numpy as jnp
from jax.experimental import pallas as pl


def kernel(table_features, table_onehot, table_dense_emb, column_statistic, filter_features, join_features, src_c2t, dst_c2t, src_t2c, dst_t2c, src_c2f, dst_c2f, src_c2j, dst_c2j, src_f2c, dst_f2c, src_j2c, dst_j2c, params):
    raise NotImplementedError("write your pallas kernel here")


# speedup vs baseline: 4.5741x; 4.5741x over previous
import functools

import jax
import jax.numpy as jnp
from jax import lax
from jax.experimental import pallas as pl
from jax.experimental.pallas import tpu as pltpu
from jax.experimental.pallas import tpu_sc as plsc

H = 8
LORA_SCALE = 2.0
BE = 128
ZR = 256
F32 = jnp.float32


def _rup(x, m):
    return (x + m - 1) // m * m


def _dot_nt(x, w):
    return lax.dot_general(x, w, (((1,), (1,)), ((), ())),
                           preferred_element_type=F32)


def _lora_body(x_ref, w_ref, a_ref, b_ref, o_ref):
    weff = w_ref[...] + LORA_SCALE * jnp.dot(
        b_ref[...], a_ref[...], preferred_element_type=F32)
    o_ref[...] = _dot_nt(x_ref[...], weff)


def _lora(x, p):
    m, k = x.shape
    n = p['W'].shape[0]
    r = p['A'].shape[0]
    bm = 1024
    return pl.pallas_call(
        _lora_body,
        grid=(pl.cdiv(m, bm),),
        in_specs=[
            pl.BlockSpec((bm, k), lambda i: (i, 0)),
            pl.BlockSpec((n, k), lambda i: (0, 0)),
            pl.BlockSpec((r, k), lambda i: (0, 0)),
            pl.BlockSpec((n, r), lambda i: (0, 0)),
        ],
        out_specs=pl.BlockSpec((bm, n), lambda i: (i, 0)),
        out_shape=jax.ShapeDtypeStruct((m, n), F32),
    )(x, p['W'], p['A'], p['B'])


def _att_vec(w, a, out_dim, k):
    w3 = w.reshape(H, out_dim, k)
    v = (a[:, :, None] * w3).sum(axis=1)
    return jnp.concatenate([v, jnp.zeros((16 - H, k), F32)], axis=0)


def _proj_body(x_ref, w_ref, al_ref, hs_ref, el_ref, *, out_dim, k):
    x = x_ref[...]
    w = w_ref[...]
    hs_ref[...] = _dot_nt(x, w)
    el_ref[...] = _dot_nt(x, _att_vec(w, al_ref[...], out_dim, k))


def _proj(x, w, al, out_dim):
    m, k = x.shape
    ho = w.shape[0]
    bm = 1024
    return pl.pallas_call(
        functools.partial(_proj_body, out_dim=out_dim, k=k),
        grid=(pl.cdiv(m, bm),),
        in_specs=[
            pl.BlockSpec((bm, k), lambda i: (i, 0)),
            pl.BlockSpec((ho, k), lambda i: (0, 0)),
            pl.BlockSpec((H, out_dim), lambda i: (0, 0)),
        ],
        out_specs=[
            pl.BlockSpec((bm, ho), lambda i: (i, 0)),
            pl.BlockSpec((bm, 16), lambda i: (i, 0)),
        ],
        out_shape=[
            jax.ShapeDtypeStruct((m, ho), F32),
            jax.ShapeDtypeStruct((m, 16), F32),
        ],
    )(x, w, al)


def _er_body(x_ref, w_ref, ar_ref, er_ref, *, out_dim, k):
    er_ref[...] = _dot_nt(x_ref[...], _att_vec(w_ref[...], ar_ref[...], out_dim, k))


def _er(x, w, ar, out_dim):
    m, k = x.shape
    ho = w.shape[0]
    bm = 1024
    return pl.pallas_call(
        functools.partial(_er_body, out_dim=out_dim, k=k),
        grid=(pl.cdiv(m, bm),),
        in_specs=[
            pl.BlockSpec((bm, k), lambda i: (i, 0)),
            pl.BlockSpec((ho, k), lambda i: (0, 0)),
            pl.BlockSpec((H, out_dim), lambda i: (0, 0)),
        ],
        out_specs=pl.BlockSpec((bm, 16), lambda i: (i, 0)),
        out_shape=jax.ShapeDtypeStruct((m, 16), F32),
    )(x, w, ar)


def _combine_body(agg_ref, den_ref, dst_ref, o_ref, *, nchl, out_dim):
    cph = out_dim // 16
    den = den_ref[...]
    acc = None
    for h in range(H):
        d = den[:, h:h + 1]
        safe = jnp.where(d > 0.0, d, 1.0)
        head = jnp.concatenate(
            [agg_ref[h * cph + m_] for m_ in range(cph)], axis=1)
        t = head / safe
        acc = t if acc is None else acc + t
    o_ref[...] = acc * (1.0 / H) + dst_ref[...]


def _combine(agg, den, dstp, out_dim):
    nchl, ndp, _ = agg.shape
    bn = 512
    return pl.pallas_call(
        functools.partial(_combine_body, nchl=nchl, out_dim=out_dim),
        grid=(ndp // bn,),
        in_specs=[
            pl.BlockSpec((nchl, bn, 16), lambda i: (0, i, 0)),
            pl.BlockSpec((bn, 16), lambda i: (i, 0)),
            pl.BlockSpec((bn, out_dim), lambda i: (i, 0)),
        ],
        out_specs=pl.BlockSpec((bn, out_dim), lambda i: (i, 0)),
        out_shape=jax.ShapeDtypeStruct((ndp, out_dim), F32),
    )(agg, den, dstp)


@functools.lru_cache(maxsize=None)
def _sc_edge(nsrc, ndp, nchl, out_dim, epad):
    nch_sc = nchl // 2
    cph = out_dim // 16
    et = epad // 16
    kb = et // BE
    rpt = ndp // 16
    nz = rpt // ZR
    rpt32 = ndp // 32
    ndz = rpt32 // 128
    mesh = plsc.VectorSubcoreMesh(core_axis_name="c", subcore_axis_name="s")

    def body(sidx_hbm, didx_hbm, el_hbm, er_hbm, hsf_hbm,
             agg_hbm, den_hbm, eet_hbm,
             shared, zero_b, sidx_b, didx_b, idx_b, eeh_b,
             el_b, er_b, eer_b, eet_b, hs_b, msg_b, sem):
        c = lax.axis_index("c")
        s = lax.axis_index("s")
        iota = lax.iota(jnp.int32, 16)
        lo8 = iota < 8

        def zfill(i, _):
            plsc.store_scatter(zero_b, [jnp.full((16,), i, jnp.int32), iota],
                               jnp.zeros((16,), F32))
            return 0
        lax.fori_loop(0, ZR, zfill, 0)

        def zcp(j, _):
            pltpu.sync_copy(zero_b, shared.at[pl.ds(s * rpt + j * ZR, ZR)])
            return 0
        lax.fori_loop(0, nz, zcp, 0)
        plsc.subcore_barrier()

        def p0_block(b, _):
            base = s * et + b * BE
            pltpu.sync_copy(sidx_hbm.at[pl.ds(base, BE)], sidx_b)
            pltpu.sync_copy(didx_hbm.at[pl.ds(base, BE)], didx_b)
            pltpu.async_copy(el_hbm.at[sidx_b], el_b, sem).wait()
            pltpu.async_copy(er_hbm.at[didx_b], er_b, sem).wait()

            def edge(e, _):
                ev = jnp.full((16,), e, jnp.int32)
                x = (plsc.load_gather(el_b, [ev, iota])
                     + plsc.load_gather(er_b, [ev, iota]))
                x = jnp.where(x >= 0.0, x, 0.2 * x)
                ee = jnp.exp(x)
                plsc.store_scatter(eer_b, [ev, iota], ee)
                plsc.store_scatter(eet_b, [iota, ev], ee, mask=lo8)
                return 0
            lax.fori_loop(0, BE, edge, 0)
            pltpu.sync_copy(eer_b, shared.at[didx_b], add=True)
            pltpu.sync_copy(eet_b, eet_hbm.at[c, :, pl.ds(base, BE)])
            return 0
        lax.fori_loop(0, kb, p0_block, 0)
        plsc.subcore_barrier()

        wid = c * 16 + s
        def dcp(j, _):
            r0 = wid * rpt32 + j * 128
            pltpu.sync_copy(shared.at[pl.ds(r0, 128)],
                            den_hbm.at[pl.ds(r0, 128), :])
            return 0
        lax.fori_loop(0, ndz, dcp, 0)
        plsc.subcore_barrier()

        def do_round(r, _):
            ch = c * nch_sc + r
            h = ch // cph

            def zcp2(j, _):
                pltpu.sync_copy(zero_b, shared.at[pl.ds(s * rpt + j * ZR, ZR)])
                return 0
            lax.fori_loop(0, nz, zcp2, 0)
            plsc.subcore_barrier()

            ch_vec = jnp.full((16,), ch, jnp.int32)

            def blk(b, _):
                base = s * et + b * BE
                pltpu.sync_copy(sidx_hbm.at[pl.ds(base, BE)], sidx_b)
                pltpu.sync_copy(didx_hbm.at[pl.ds(base, BE)], didx_b)
                pltpu.sync_copy(eet_hbm.at[c, h, pl.ds(base, BE)], eeh_b)

                def gidx(g, _):
                    v = sidx_b[pl.ds(g * 16, 16)]
                    idx_b[pl.ds(g * 16, 16)] = v * nchl + ch_vec
                    return 0
                lax.fori_loop(0, BE // 16, gidx, 0)
                pltpu.async_copy(hsf_hbm.at[idx_b], hs_b, sem).wait()

                def grp(g, _):
                    e_ids = g * 16 + iota
                    ee_vec = eeh_b[pl.ds(g * 16, 16)]
                    for j in range(16):
                        jv = jnp.full((16,), j, jnp.int32)
                        col = plsc.load_gather(hs_b, [e_ids, jv])
                        plsc.store_scatter(msg_b, [e_ids, jv], col * ee_vec)
                    return 0
                lax.fori_loop(0, BE // 16, grp, 0)
                pltpu.sync_copy(msg_b, shared.at[didx_b], add=True)
                return 0
            lax.fori_loop(0, kb, blk, 0)
            plsc.subcore_barrier()

            def ocp(j, _):
                r0 = s * rpt + j * ZR
                pltpu.sync_copy(shared.at[pl.ds(r0, ZR)],
                                agg_hbm.at[ch, pl.ds(r0, ZR), :])
                return 0
            lax.fori_loop(0, nz, ocp, 0)
            plsc.subcore_barrier()
            return 0
        lax.fori_loop(0, nch_sc, do_round, 0)

    return pl.kernel(
        body,
        out_type=[
            jax.ShapeDtypeStruct((nchl, ndp, 16), F32),
            jax.ShapeDtypeStruct((ndp, 16), F32),
            jax.ShapeDtypeStruct((2, 8, epad), F32),
        ],
        mesh=mesh,
        compiler_params=pltpu.CompilerParams(needs_layout_passes=False,
                                             use_tc_tiling_on_sc=False),
        scratch_types=[
            pltpu.VMEM_SHARED((ndp, 16), F32),
            pltpu.VMEM((ZR, 16), F32),
            pltpu.VMEM((BE,), jnp.int32),
            pltpu.VMEM((BE,), jnp.int32),
            pltpu.VMEM((BE,), jnp.int32),
            pltpu.VMEM((BE,), F32),
            pltpu.VMEM((BE, 16), F32),
            pltpu.VMEM((BE, 16), F32),
            pltpu.VMEM((BE, 16), F32),
            pltpu.VMEM((8, BE), F32),
            pltpu.VMEM((BE, 16), F32),
            pltpu.VMEM((BE, 16), F32),
            pltpu.SemaphoreType.DMA,
        ],
    )


def _gat(src_feat, dst_feat, sidx, didx, gp):
    nsrc, _ = src_feat.shape
    ndst, ddst = dst_feat.shape
    ho = gp['Ws'].shape[0]
    out_dim = ho // H
    nchl = ho // 16
    ndp = _rup(ndst + 16, 4096)
    e = sidx.shape[0]
    epad = _rup(e, 2048)

    hs, el16 = _proj(src_feat, gp['Ws'], gp['al'], out_dim)
    dstp = jnp.pad(dst_feat, ((0, ndp - ndst), (0, 0)))
    er16 = _er(dstp, gp['Wd'], gp['ar'], out_dim)
    sidxp = jnp.pad(sidx, (0, epad - e))
    didxp = jnp.pad(didx, (0, epad - e), constant_values=ndst)
    hsf = hs.reshape(nsrc * nchl, 16)

    agg, den, _ = _sc_edge(nsrc, ndp, nchl, out_dim, epad)(
        sidxp, didxp, el16, er16, hsf)
    return _combine(agg, den, dstp, out_dim)[:ndst]


def kernel(table_features, table_onehot, table_dense_emb, column_statistic,
           filter_features, join_features,
           src_c2t, dst_c2t, src_t2c, dst_t2c, src_c2f, dst_c2f,
           src_c2j, dst_c2j, src_f2c, dst_f2c, src_j2c, dst_j2c, params):
    p = params
    tab = _lora(jnp.concatenate(
        [table_features, table_onehot, table_dense_emb], axis=-1), p['fc_table'])
    col = _lora(column_statistic, p['fc_column'])
    fil = _lora(filter_features, p['fc_filter'])
    joi = _lora(join_features, p['fc_join'])
    t1 = _gat(col, tab, src_c2t, dst_c2t, p['c2t1'])
    c1 = _gat(t1, col, src_t2c, dst_t2c, p['t2c1'])
    f1 = _gat(c1, fil, src_c2f, dst_c2f, p['c2f1'])
    j1 = _gat(c1, joi, src_c2j, dst_c2j, p['c2j1'])
    c2_f = _gat(f1, c1, src_f2c, dst_f2c, p['f2c1'])
    c2_j = _gat(j1, c1, src_j2c, dst_j2c, p['j2c1'])
    t2 = _gat((c2_f + c2_j) * 0.5, t1, src_c2t, dst_c2t, p['c2t2'])
    return _lora(jnp.concatenate([t2, table_features], axis=-1), p['fc_out'])


# speedup vs baseline: 7.2024x; 1.5746x over previous
import functools

import jax
import jax.numpy as jnp
from jax import lax
from jax.experimental import pallas as pl
from jax.experimental.pallas import tpu as pltpu
from jax.experimental.pallas import tpu_sc as plsc

H = 8
LORA_SCALE = 2.0
BE = 128
ZR = 256
F32 = jnp.float32


def _rup(x, m):
    return (x + m - 1) // m * m


def _dot_nt(x, w):
    return lax.dot_general(x, w, (((1,), (1,)), ((), ())),
                           preferred_element_type=F32)


def _lora_body(x_ref, w_ref, a_ref, b_ref, o_ref):
    weff = w_ref[...] + LORA_SCALE * jnp.dot(
        b_ref[...], a_ref[...], preferred_element_type=F32)
    o_ref[...] = _dot_nt(x_ref[...], weff)


def _lora(x, p):
    m, k = x.shape
    n = p['W'].shape[0]
    r = p['A'].shape[0]
    bm = 1024
    return pl.pallas_call(
        _lora_body,
        grid=(pl.cdiv(m, bm),),
        in_specs=[
            pl.BlockSpec((bm, k), lambda i: (i, 0)),
            pl.BlockSpec((n, k), lambda i: (0, 0)),
            pl.BlockSpec((r, k), lambda i: (0, 0)),
            pl.BlockSpec((n, r), lambda i: (0, 0)),
        ],
        out_specs=pl.BlockSpec((bm, n), lambda i: (i, 0)),
        out_shape=jax.ShapeDtypeStruct((m, n), F32),
    )(x, p['W'], p['A'], p['B'])


def _att_vec(w, a, out_dim, k):
    w3 = w.reshape(H, out_dim, k)
    v = (a[:, :, None] * w3).sum(axis=1)
    return jnp.concatenate([v, jnp.zeros((16 - H, k), F32)], axis=0)


def _proj_body(x_ref, w_ref, al_ref, hs_ref, el_ref, *, out_dim, k):
    x = x_ref[...]
    w = w_ref[...]
    hs_ref[...] = _dot_nt(x, w)
    el_ref[...] = _dot_nt(x, _att_vec(w, al_ref[...], out_dim, k))


def _proj(x, w, al, out_dim):
    m, k = x.shape
    ho = w.shape[0]
    bm = 1024
    return pl.pallas_call(
        functools.partial(_proj_body, out_dim=out_dim, k=k),
        grid=(pl.cdiv(m, bm),),
        in_specs=[
            pl.BlockSpec((bm, k), lambda i: (i, 0)),
            pl.BlockSpec((ho, k), lambda i: (0, 0)),
            pl.BlockSpec((H, out_dim), lambda i: (0, 0)),
        ],
        out_specs=[
            pl.BlockSpec((bm, ho), lambda i: (i, 0)),
            pl.BlockSpec((bm, 16), lambda i: (i, 0)),
        ],
        out_shape=[
            jax.ShapeDtypeStruct((m, ho), F32),
            jax.ShapeDtypeStruct((m, 16), F32),
        ],
    )(x, w, al)


def _er_body(x_ref, w_ref, ar_ref, er_ref, *, out_dim, k):
    er_ref[...] = _dot_nt(x_ref[...], _att_vec(w_ref[...], ar_ref[...], out_dim, k))


def _er(x, w, ar, out_dim):
    m, k = x.shape
    ho = w.shape[0]
    bm = 1024
    return pl.pallas_call(
        functools.partial(_er_body, out_dim=out_dim, k=k),
        grid=(pl.cdiv(m, bm),),
        in_specs=[
            pl.BlockSpec((bm, k), lambda i: (i, 0)),
            pl.BlockSpec((ho, k), lambda i: (0, 0)),
            pl.BlockSpec((H, out_dim), lambda i: (0, 0)),
        ],
        out_specs=pl.BlockSpec((bm, 16), lambda i: (i, 0)),
        out_shape=jax.ShapeDtypeStruct((m, 16), F32),
    )(x, w, ar)


def _combine_body(agg_ref, den_ref, dst_ref, o_ref, *, nchl, out_dim):
    cph = out_dim // 16
    den = den_ref[...]
    acc = None
    for h in range(H):
        d = den[:, h:h + 1]
        safe = jnp.where(d > 0.0, d, 1.0)
        head = jnp.concatenate(
            [agg_ref[h * cph + m_] for m_ in range(cph)], axis=1)
        t = head / safe
        acc = t if acc is None else acc + t
    o_ref[...] = acc * (1.0 / H) + dst_ref[...]


def _combine(agg, den, dstp, out_dim):
    nchl, ndp, _ = agg.shape
    bn = 512
    return pl.pallas_call(
        functools.partial(_combine_body, nchl=nchl, out_dim=out_dim),
        grid=(ndp // bn,),
        in_specs=[
            pl.BlockSpec((nchl, bn, 16), lambda i: (0, i, 0)),
            pl.BlockSpec((bn, 16), lambda i: (i, 0)),
            pl.BlockSpec((bn, out_dim), lambda i: (i, 0)),
        ],
        out_specs=pl.BlockSpec((bn, out_dim), lambda i: (i, 0)),
        out_shape=jax.ShapeDtypeStruct((ndp, out_dim), F32),
    )(agg, den, dstp)


_SC_PARAMS = pltpu.CompilerParams(needs_layout_passes=False,
                                  use_tc_tiling_on_sc=False)
_MESH = plsc.VectorSubcoreMesh(core_axis_name="c", subcore_axis_name="s")


def _zfill(zero_b, iota):
    def zf(i, _):
        plsc.store_scatter(zero_b, [jnp.full((16,), i, jnp.int32), iota],
                           jnp.zeros((16,), F32))
        return 0
    lax.fori_loop(0, 128, zf, 0)


def _zero_shared(shared, zero_b, sz, base, nz):
    def zi(j, _):
        pltpu.async_copy(zero_b, shared.at[pl.ds(base + j * 128, 128)], sz)
        return 0
    lax.fori_loop(0, nz, zi, 0)

    def zw(j, _):
        pltpu.make_async_copy(zero_b,
                              shared.at[pl.ds(base + j * 128, 128)],
                              sz).wait()
        return 0
    lax.fori_loop(0, nz, zw, 0)


@functools.lru_cache(maxsize=None)
def _sc_phase0(nsrc, ndp, epad):
    et = epad // 16
    kb = et // BE
    kb2 = kb // 2
    rpt = ndp // 16
    nz = rpt // 128

    def body(sidx_hbm, didx_hbm, el_hbm, er_hbm,
             den_hbm, eet_hbm,
             shared, zero_b, sidx_v, didx_v,
             el_b, er_b, eer_b, eet_b, sg0, sg1, sz):
        c = lax.axis_index("c")
        s = lax.axis_index("s")
        iota = lax.iota(jnp.int32, 16)
        lo8 = iota < 8
        sg = (sg0, sg1)

        _zfill(zero_b, iota)
        pltpu.sync_copy(sidx_hbm.at[pl.ds(s * kb, kb), :], sidx_v)
        pltpu.sync_copy(didx_hbm.at[pl.ds(s * kb, kb), :], didx_v)
        _zero_shared(shared, zero_b, sz, s * rpt, nz)
        plsc.subcore_barrier()

        def p0_issue(b, slot):
            pltpu.async_copy(el_hbm.at[sidx_v.at[b]], el_b.at[slot], sg[slot])
            pltpu.async_copy(er_hbm.at[didx_v.at[b]], er_b.at[slot], sg[slot])

        def p0_wait(b, slot):
            pltpu.make_async_copy(el_hbm.at[sidx_v.at[b]],
                                  el_b.at[slot], sg[slot]).wait()
            pltpu.make_async_copy(er_hbm.at[didx_v.at[b]],
                                  er_b.at[slot], sg[slot]).wait()

        def p0_process(b, slot):
            def edge(e, _):
                ev = jnp.full((16,), e, jnp.int32)
                x = (plsc.load_gather(el_b.at[slot], [ev, iota])
                     + plsc.load_gather(er_b.at[slot], [ev, iota]))
                x = jnp.where(x >= 0.0, x, 0.2 * x)
                ee = jnp.exp(x)
                plsc.store_scatter(eer_b, [ev, iota], ee)
                plsc.store_scatter(eet_b, [iota, ev], ee, mask=lo8)
                return 0
            lax.fori_loop(0, BE, edge, 0)
            pltpu.sync_copy(eer_b, shared.at[didx_v.at[b]], add=True)
            pltpu.sync_copy(eet_b,
                            eet_hbm.at[c, :, pl.ds(s * et + b * BE, BE)])

        p0_issue(0, 0)

        def p0_loop(b2, _):
            b = 2 * b2
            p0_issue(b + 1, 1)
            p0_wait(b, 0)
            p0_process(b, 0)

            @pl.when(b2 + 1 < kb2)
            def _():
                p0_issue(b + 2, 0)
            p0_wait(b + 1, 1)
            p0_process(b + 1, 1)
            return 0
        lax.fori_loop(0, kb2, p0_loop, 0)
        plsc.subcore_barrier()

        @pl.when(c == 0)
        def _():
            def di(j, _):
                r0 = s * rpt + j * 128
                pltpu.async_copy(shared.at[pl.ds(r0, 128)],
                                 den_hbm.at[pl.ds(r0, 128), :], sz)
                return 0
            lax.fori_loop(0, nz, di, 0)

            def dw(j, _):
                r0 = s * rpt + j * 128
                pltpu.make_async_copy(shared.at[pl.ds(r0, 128)],
                                     den_hbm.at[pl.ds(r0, 128), :],
                                     sz).wait()
                return 0
            lax.fori_loop(0, nz, dw, 0)

    return pl.kernel(
        body,
        out_type=[
            jax.ShapeDtypeStruct((ndp, 16), F32),
            jax.ShapeDtypeStruct((2, 8, epad), F32),
        ],
        mesh=_MESH,
        compiler_params=_SC_PARAMS,
        scratch_types=[
            pltpu.VMEM_SHARED((ndp, 16), F32),
            pltpu.VMEM((128, 16), F32),
            pltpu.VMEM((epad // BE // 16, BE), jnp.int32),
            pltpu.VMEM((epad // BE // 16, BE), jnp.int32),
            pltpu.VMEM((2, BE, 16), F32),
            pltpu.VMEM((2, BE, 16), F32),
            pltpu.VMEM((BE, 16), F32),
            pltpu.VMEM((8, BE), F32),
            pltpu.SemaphoreType.DMA,
            pltpu.SemaphoreType.DMA,
            pltpu.SemaphoreType.DMA,
        ],
    )


@functools.lru_cache(maxsize=None)
def _sc_rounds(nsrc, ndp, nchl, out_dim, epad):
    nch_sc = nchl // 2
    cph = out_dim // 16
    et = epad // 16
    kb = et // BE
    kb2 = kb // 2
    rpt = ndp // 16
    nz = rpt // 128

    def body(sidx_hbm, didx_hbm, eet_hbm, hsf_hbm,
             agg_hbm,
             shared, zero_b, idx_v, didx_v, eeh_v,
             hs_b, msg_b, sg0, sg1, ss0, ss1, sz):
        c = lax.axis_index("c")
        s = lax.axis_index("s")
        iota = lax.iota(jnp.int32, 16)
        sg = (sg0, sg1)
        ss = (ss0, ss1)

        _zfill(zero_b, iota)
        pltpu.sync_copy(sidx_hbm.at[pl.ds(s * kb, kb), :], idx_v)
        pltpu.sync_copy(didx_hbm.at[pl.ds(s * kb, kb), :], didx_v)
        ch0 = jnp.full((16,), c * nch_sc, jnp.int32)

        def tidx(g, _):
            b = g // (BE // 16)
            o = (g % (BE // 16)) * 16
            idx_v[b, pl.ds(o, 16)] = idx_v[b, pl.ds(o, 16)] * nchl + ch0
            return 0
        lax.fori_loop(0, kb * (BE // 16), tidx, 0)
        _zero_shared(shared, zero_b, sz, s * rpt, nz)
        plsc.subcore_barrier()

        def do_round(r, _):
            ch = c * nch_sc + r
            h = ch // cph
            pltpu.sync_copy(eet_hbm.at[c, h, pl.ds(s * et, et)], eeh_v)

            def r_issue(b, slot):
                pltpu.async_copy(hsf_hbm.at[idx_v.at[b]], hs_b.at[slot],
                                 sg[slot])

            def r_wait(b, slot):
                pltpu.make_async_copy(hsf_hbm.at[idx_v.at[b]],
                                      hs_b.at[slot], sg[slot]).wait()

            def drain(slot):
                pltpu.make_async_copy(msg_b.at[slot],
                                      shared.at[didx_v.at[0]],
                                      ss[slot]).wait()

            def r_process(b, b2, slot):
                @pl.when(b2 > 0)
                def _():
                    drain(slot)

                def grp(g, _):
                    e_ids = g * 16 + iota
                    ee_vec = eeh_v[pl.ds(b * BE + g * 16, 16)]
                    for j in range(16):
                        jv = jnp.full((16,), j, jnp.int32)
                        col = plsc.load_gather(hs_b.at[slot], [e_ids, jv])
                        plsc.store_scatter(msg_b.at[slot], [e_ids, jv],
                                           col * ee_vec)
                    return 0
                lax.fori_loop(0, BE // 16, grp, 0)
                pltpu.async_copy(msg_b.at[slot], shared.at[didx_v.at[b]],
                                 ss[slot], add=True)

            r_issue(0, 0)

            def r_loop(b2, _):
                b = 2 * b2
                r_issue(b + 1, 1)
                r_wait(b, 0)
                r_process(b, b2, 0)

                @pl.when(b2 + 1 < kb2)
                def _():
                    r_issue(b + 2, 0)
                r_wait(b + 1, 1)
                r_process(b + 1, b2, 1)
                return 0
            lax.fori_loop(0, kb2, r_loop, 0)
            drain(0)
            drain(1)
            plsc.subcore_barrier()

            def oi(j, _):
                r0 = s * rpt + j * 128
                pltpu.async_copy(shared.at[pl.ds(r0, 128)],
                                 agg_hbm.at[ch, pl.ds(r0, 128), :], ss0)
                return 0
            lax.fori_loop(0, nz, oi, 0)

            def ow(j, _):
                r0 = s * rpt + j * 128
                pltpu.make_async_copy(shared.at[pl.ds(r0, 128)],
                                     agg_hbm.at[ch, pl.ds(r0, 128), :],
                                     ss0).wait()
                pltpu.async_copy(zero_b, shared.at[pl.ds(r0, 128)], sz)
                return 0
            lax.fori_loop(0, nz, ow, 0)

            def zw(j, _):
                pltpu.make_async_copy(zero_b,
                                      shared.at[pl.ds(s * rpt + j * 128, 128)],
                                      sz).wait()
                return 0
            lax.fori_loop(0, nz, zw, 0)

            def uidx(g, _):
                b = g // (BE // 16)
                o = (g % (BE // 16)) * 16
                idx_v[b, pl.ds(o, 16)] = idx_v[b, pl.ds(o, 16)] + 1
                return 0
            lax.fori_loop(0, kb * (BE // 16), uidx, 0)
            plsc.subcore_barrier()
            return 0
        lax.fori_loop(0, nch_sc, do_round, 0)

    return pl.kernel(
        body,
        out_type=jax.ShapeDtypeStruct((nchl, ndp, 16), F32),
        mesh=_MESH,
        compiler_params=_SC_PARAMS,
        scratch_types=[
            pltpu.VMEM_SHARED((ndp, 16), F32),
            pltpu.VMEM((128, 16), F32),
            pltpu.VMEM((epad // BE // 16, BE), jnp.int32),
            pltpu.VMEM((epad // BE // 16, BE), jnp.int32),
            pltpu.VMEM((epad // 16,), F32),
            pltpu.VMEM((2, BE, 16), F32),
            pltpu.VMEM((2, BE, 16), F32),
            pltpu.SemaphoreType.DMA,
            pltpu.SemaphoreType.DMA,
            pltpu.SemaphoreType.DMA,
            pltpu.SemaphoreType.DMA,
            pltpu.SemaphoreType.DMA,
        ],
    )


def _gat(src_feat, dst_feat, sidx, didx, gp):
    nsrc, _ = src_feat.shape
    ndst, ddst = dst_feat.shape
    ho = gp['Ws'].shape[0]
    out_dim = ho // H
    nchl = ho // 16
    ndp = _rup(ndst + 16, 2048)
    e = sidx.shape[0]
    epad = _rup(e, 4096)

    hs, el16 = _proj(src_feat, gp['Ws'], gp['al'], out_dim)
    dstp = jnp.pad(dst_feat, ((0, ndp - ndst), (0, 0)))
    er16 = _er(dstp, gp['Wd'], gp['ar'], out_dim)
    sidxp = jnp.pad(sidx, (0, epad - e)).reshape(epad // BE, BE)
    didxp = jnp.pad(didx, (0, epad - e),
                    constant_values=ndst).reshape(epad // BE, BE)
    hsf = hs.reshape(nsrc * nchl, 16)

    den, eet = _sc_phase0(nsrc, ndp, epad)(sidxp, didxp, el16, er16)
    agg = _sc_rounds(nsrc, ndp, nchl, out_dim, epad)(sidxp, didxp, eet, hsf)
    return _combine(agg, den, dstp, out_dim)[:ndst]


def kernel(table_features, table_onehot, table_dense_emb, column_statistic,
           filter_features, join_features,
           src_c2t, dst_c2t, src_t2c, dst_t2c, src_c2f, dst_c2f,
           src_c2j, dst_c2j, src_f2c, dst_f2c, src_j2c, dst_j2c, params):
    p = params
    tab = _lora(jnp.concatenate(
        [table_features, table_onehot, table_dense_emb], axis=-1), p['fc_table'])
    col = _lora(column_statistic, p['fc_column'])
    fil = _lora(filter_features, p['fc_filter'])
    joi = _lora(join_features, p['fc_join'])
    t1 = _gat(col, tab, src_c2t, dst_c2t, p['c2t1'])
    c1 = _gat(t1, col, src_t2c, dst_t2c, p['t2c1'])
    f1 = _gat(c1, fil, src_c2f, dst_c2f, p['c2f1'])
    j1 = _gat(c1, joi, src_c2j, dst_c2j, p['c2j1'])
    c2_f = _gat(f1, c1, src_f2c, dst_f2c, p['f2c1'])
    c2_j = _gat(j1, c1, src_j2c, dst_j2c, p['j2c1'])
    t2 = _gat((c2_f + c2_j) * 0.5, t1, src_c2t, dst_c2t, p['c2t2'])
    return _lora(jnp.concatenate([t2, table_features], axis=-1), p['fc_out'])
